# Initial kernel scaffold; baseline (speedup 1.0000x reference)
import jax
import jax.numpy as jnp
from jax.experimental import pallas as pl


def kernel(x, t, Wg, bg, W1, b1, W2, b2, W3, b3, Wt):
    raise NotImplementedError("write your pallas kernel here")


# speedup vs baseline: 1.1951x; 1.1951x over previous
import jax
import jax.numpy as jnp
from jax.experimental import pallas as pl
from jax.experimental.pallas import tpu as pltpu

_NEIGHBORS = (0, 5, 11, 17)
_TIME_LEN = 9
_E = 16
_D = 128
_NB = 4
_FLAT = _TIME_LEN * _NB * _D
_H = 512
_KL_W = 0.01


def _moe_kernel(t_ref, x_ref, wg_ref, bg_ref, w1_ref, b1_ref, w2_ref, b2_ref,
                w3_ref, b3_ref, wt_ref, loss_ref, idx_ref,
                flat_scr, flatb_scr, g_scr, mse_scr, tgt_scr, sem):
    e = pl.program_id(0)
    dt_half = _TIME_LEN // 2

    @pl.when(e == 0)
    def _gather_and_route():
        t0 = t_ref[0] - dt_half
        copies = []
        for ti in range(_TIME_LEN):
            for nb in range(_NB):
                j = _NEIGHBORS[nb]
                k = ti * _NB + nb
                c = pltpu.make_async_copy(
                    x_ref.at[:, t0 + ti, j, :],
                    flat_scr.at[:, pl.ds(k * _D, _D)],
                    sem,
                )
                c.start()
                copies.append(c)
        for c in copies:
            c.wait()

        flat = flat_scr[...]
        flatb = flat.astype(jnp.bfloat16)
        flatb_scr[...] = flatb

        logits = jnp.dot(flatb, wg_ref[...].astype(jnp.bfloat16),
                         preferred_element_type=jnp.float32) + bg_ref[...]
        m = jnp.max(logits, axis=-1, keepdims=True)
        ex = jnp.exp(logits - m)
        g = ex / jnp.sum(ex, axis=-1, keepdims=True)
        g_scr[...] = g

        lane = jax.lax.broadcasted_iota(jnp.int32, logits.shape, 1)
        is_max = logits == jnp.max(logits, axis=-1, keepdims=True)
        idx = jnp.min(jnp.where(is_max, lane, _E), axis=-1)
        idx_ref[0, :] = idx

        center = flat_scr[:, pl.ds(dt_half * _NB * _D, _NB * _D)]
        tgt_scr[...] = jnp.dot(center.astype(jnp.bfloat16),
                               wt_ref[...].astype(jnp.bfloat16),
                               preferred_element_type=jnp.float32)
        mse_scr[...] = jnp.zeros_like(mse_scr)

    flatb = flatb_scr[...]
    h = jnp.dot(flatb, w1_ref[0].astype(jnp.bfloat16),
                preferred_element_type=jnp.float32) + b1_ref[0]
    h = jnp.maximum(h, 0.0)
    h = jnp.dot(h.astype(jnp.bfloat16), w2_ref[0].astype(jnp.bfloat16),
                preferred_element_type=jnp.float32) + b2_ref[0]
    h = jnp.maximum(h, 0.0)
    y = jnp.dot(h.astype(jnp.bfloat16), w3_ref[0].astype(jnp.bfloat16),
                preferred_element_type=jnp.float32) + b3_ref[0]
    mse_e = jnp.mean((y - tgt_scr[...]) ** 2, axis=-1)
    onehot = (jax.lax.broadcasted_iota(jnp.int32, (1, _E), 1) == e
              ).astype(jnp.float32)
    mse_scr[...] += mse_e[:, None] * onehot

    @pl.when(e == _E - 1)
    def _finalize():
        g = g_scr[...]
        B = g.shape[0]
        weighted = jnp.sum(g * mse_scr[...]) / B
        usage = jnp.sum(g, axis=0, keepdims=True) / B
        kl = jnp.sum(usage * (jnp.log(usage + 1e-9) - jnp.log(1.0 / _E)))
        loss_ref[...] = jnp.reshape(weighted + _KL_W * kl, (1, 1))


def kernel(x, t, Wg, bg, W1, b1, W2, b2, W3, b3, Wt):
    B = x.shape[0]
    t_arr = jnp.asarray(t, jnp.int32).reshape(1)
    bg2 = bg.reshape(1, _E)
    b1r = b1.reshape(_E, 1, _H)
    b2r = b2.reshape(_E, 1, _H)
    b3r = b3.reshape(_E, 1, _D)

    loss, idx = pl.pallas_call(
        _moe_kernel,
        grid=(_E,),
        in_specs=[
            pl.BlockSpec(memory_space=pltpu.SMEM),
            pl.BlockSpec(memory_space=pltpu.HBM),
            pl.BlockSpec((_FLAT, _E), lambda e: (0, 0)),
            pl.BlockSpec((1, _E), lambda e: (0, 0)),
            pl.BlockSpec((1, _FLAT, _H), lambda e: (e, 0, 0)),
            pl.BlockSpec((1, 1, _H), lambda e: (e, 0, 0)),
            pl.BlockSpec((1, _H, _H), lambda e: (e, 0, 0)),
            pl.BlockSpec((1, 1, _H), lambda e: (e, 0, 0)),
            pl.BlockSpec((1, _H, _D), lambda e: (e, 0, 0)),
            pl.BlockSpec((1, 1, _D), lambda e: (e, 0, 0)),
            pl.BlockSpec((_NB * _D, _D), lambda e: (0, 0)),
        ],
        out_specs=[
            pl.BlockSpec((1, 1), lambda e: (0, 0)),
            pl.BlockSpec((1, B), lambda e: (0, 0)),
        ],
        out_shape=[
            jax.ShapeDtypeStruct((1, 1), jnp.float32),
            jax.ShapeDtypeStruct((1, B), jnp.int32),
        ],
        scratch_shapes=[
            pltpu.VMEM((B, _FLAT), jnp.float32),
            pltpu.VMEM((B, _FLAT), jnp.bfloat16),
            pltpu.VMEM((B, _E), jnp.float32),
            pltpu.VMEM((B, _E), jnp.float32),
            pltpu.VMEM((B, _D), jnp.float32),
            pltpu.SemaphoreType.DMA,
        ],
        compiler_params=pltpu.CompilerParams(
            dimension_semantics=("arbitrary",),
        ),
    )(t_arr, x, Wg, bg2, W1, b1r, W2, b2r, W3, b3r, Wt)
    return loss.reshape(()), idx.reshape(B)


# speedup vs baseline: 1.2010x; 1.0049x over previous
import jax
import jax.numpy as jnp
from jax.experimental import pallas as pl
from jax.experimental.pallas import tpu as pltpu

_NEIGHBORS = (0, 5, 11, 17)
_TIME_LEN = 9
_E = 16
_D = 128
_NB = 4
_FLAT = _TIME_LEN * _NB * _D
_H = 512
_KL_W = 0.01


def _moe_kernel(t_ref, x_ref, wg_ref, bg_ref, w1a_ref, w1b_ref, b1_ref,
                w2_ref, b2_ref, w3_ref, b3_ref, wt_ref, loss_ref, idx_ref,
                flat_scr, flatb_scr, g_scr, mse_scr, tgt_scr, sem):
    e = pl.program_id(0)
    dt_half = _TIME_LEN // 2

    @pl.when(e == 0)
    def _gather_and_route():
        t0 = t_ref[0] - dt_half
        copies = []
        for ti in range(_TIME_LEN):
            for nb in range(_NB):
                j = _NEIGHBORS[nb]
                k = ti * _NB + nb
                c = pltpu.make_async_copy(
                    x_ref.at[:, t0 + ti, j, :],
                    flat_scr.at[:, pl.ds(k * _D, _D)],
                    sem,
                )
                c.start()
                copies.append(c)
        for c in copies:
            c.wait()

        flat = flat_scr[...]
        flatb = flat.astype(jnp.bfloat16)
        flatb_scr[...] = flatb

        logits = jnp.dot(flatb, wg_ref[...].astype(jnp.bfloat16),
                         preferred_element_type=jnp.float32) + bg_ref[...]
        m = jnp.max(logits, axis=-1, keepdims=True)
        ex = jnp.exp(logits - m)
        g = ex / jnp.sum(ex, axis=-1, keepdims=True)
        g_scr[...] = g

        lane = jax.lax.broadcasted_iota(jnp.int32, logits.shape, 1)
        is_max = logits == jnp.max(logits, axis=-1, keepdims=True)
        idx = jnp.min(jnp.where(is_max, lane, _E), axis=-1)
        idx_ref[0, :] = idx

        center = flat_scr[:, pl.ds(dt_half * _NB * _D, _NB * _D)]
        tgt_scr[...] = jnp.dot(center.astype(jnp.bfloat16),
                               wt_ref[...].astype(jnp.bfloat16),
                               preferred_element_type=jnp.float32)
        mse_scr[...] = jnp.zeros_like(mse_scr)

    flatb = flatb_scr[...]
    hh = _H // 2
    h0 = jnp.dot(flatb, w1a_ref[0].astype(jnp.bfloat16),
                 preferred_element_type=jnp.float32) + b1_ref[0, :, :hh]
    h1 = jnp.dot(flatb, w1b_ref[0].astype(jnp.bfloat16),
                 preferred_element_type=jnp.float32) + b1_ref[0, :, hh:]
    h0 = jnp.maximum(h0, 0.0).astype(jnp.bfloat16)
    h1 = jnp.maximum(h1, 0.0).astype(jnp.bfloat16)
    h = (jnp.dot(h0, w2_ref[0, :hh, :].astype(jnp.bfloat16),
                 preferred_element_type=jnp.float32)
         + jnp.dot(h1, w2_ref[0, hh:, :].astype(jnp.bfloat16),
                   preferred_element_type=jnp.float32)) + b2_ref[0]
    h = jnp.maximum(h, 0.0)
    y = jnp.dot(h.astype(jnp.bfloat16), w3_ref[0].astype(jnp.bfloat16),
                preferred_element_type=jnp.float32) + b3_ref[0]
    mse_e = jnp.mean((y - tgt_scr[...]) ** 2, axis=-1)
    onehot = (jax.lax.broadcasted_iota(jnp.int32, (1, _E), 1) == e
              ).astype(jnp.float32)
    mse_scr[...] += mse_e[:, None] * onehot

    @pl.when(e == _E - 1)
    def _finalize():
        g = g_scr[...]
        B = g.shape[0]
        weighted = jnp.sum(g * mse_scr[...]) / B
        usage = jnp.sum(g, axis=0, keepdims=True) / B
        kl = jnp.sum(usage * (jnp.log(usage + 1e-9) - jnp.log(1.0 / _E)))
        loss_ref[...] = jnp.reshape(weighted + _KL_W * kl, (1, 1))


def kernel(x, t, Wg, bg, W1, b1, W2, b2, W3, b3, Wt):
    B = x.shape[0]
    t_arr = jnp.asarray(t, jnp.int32).reshape(1)
    bg2 = bg.reshape(1, _E)
    b1r = b1.reshape(_E, 1, _H)
    b2r = b2.reshape(_E, 1, _H)
    b3r = b3.reshape(_E, 1, _D)

    loss, idx = pl.pallas_call(
        _moe_kernel,
        grid=(_E,),
        in_specs=[
            pl.BlockSpec(memory_space=pltpu.SMEM),
            pl.BlockSpec(memory_space=pltpu.HBM),
            pl.BlockSpec((_FLAT, _E), lambda e: (0, 0)),
            pl.BlockSpec((1, _E), lambda e: (0, 0)),
            pl.BlockSpec((1, _FLAT, _H // 2), lambda e: (e, 0, 0)),
            pl.BlockSpec((1, _FLAT, _H // 2), lambda e: (e, 0, 1)),
            pl.BlockSpec((1, 1, _H), lambda e: (e, 0, 0)),
            pl.BlockSpec((1, _H, _H), lambda e: (e, 0, 0)),
            pl.BlockSpec((1, 1, _H), lambda e: (e, 0, 0)),
            pl.BlockSpec((1, _H, _D), lambda e: (e, 0, 0)),
            pl.BlockSpec((1, 1, _D), lambda e: (e, 0, 0)),
            pl.BlockSpec((_NB * _D, _D), lambda e: (0, 0)),
        ],
        out_specs=[
            pl.BlockSpec((1, 1), lambda e: (0, 0)),
            pl.BlockSpec((1, B), lambda e: (0, 0)),
        ],
        out_shape=[
            jax.ShapeDtypeStruct((1, 1), jnp.float32),
            jax.ShapeDtypeStruct((1, B), jnp.int32),
        ],
        scratch_shapes=[
            pltpu.VMEM((B, _FLAT), jnp.float32),
            pltpu.VMEM((B, _FLAT), jnp.bfloat16),
            pltpu.VMEM((B, _E), jnp.float32),
            pltpu.VMEM((B, _E), jnp.float32),
            pltpu.VMEM((B, _D), jnp.float32),
            pltpu.SemaphoreType.DMA,
        ],
        compiler_params=pltpu.CompilerParams(
            dimension_semantics=("arbitrary",),
        ),
    )(t_arr, x, Wg, bg2, W1, W1, b1r, W2, b2r, W3, b3r, Wt)
    return loss.reshape(()), idx.reshape(B)
